# baseline (speedup 1.0000x reference)
import jax
import jax.numpy as jnp
from jax.experimental import pallas as pl
from jax.experimental.pallas import tpu as pltpu

B = 4
S = 4096
D = 2048
K = 512
SBLK = 128


def _imp_kernel(h_ref, o_ref):
    x = h_ref[...]
    xc = x.reshape(B, SBLK, 16, 128)
    acc = xc[:, :, 0, :] * xc[:, :, 0, :]
    for c in range(1, 16):
        acc = acc + xc[:, :, c, :] * xc[:, :, c, :]
    pr = acc.reshape(B, SBLK, 16, 8)
    q = pr[:, :, 0, :]
    for j in range(1, 16):
        q = q + pr[:, :, j, :]
    ss = (((q[..., 0] + q[..., 4]) + (q[..., 2] + q[..., 6]))
          + ((q[..., 1] + q[..., 5]) + (q[..., 3] + q[..., 7])))
    n = jnp.sqrt(ss)
    o_ref[...] = jnp.mean(n, axis=0)


def _importance(hidden_states):
    return pl.pallas_call(
        _imp_kernel,
        grid=(S // SBLK,),
        in_specs=[pl.BlockSpec((B, SBLK, D), lambda i: (0, i, 0))],
        out_specs=pl.BlockSpec((SBLK,), lambda i: (i,)),
        out_shape=jax.ShapeDtypeStruct((S,), jnp.float32),
    )(hidden_states)


def _gather_mean_kernel(idx_ref, h_ref, o_ref):
    o_ref[...] = jnp.mean(h_ref[...], axis=0)


def _gather_mean(hidden_states, topk_indices):
    h4 = hidden_states.reshape(B, S, 16, 128)
    out = pl.pallas_call(
        _gather_mean_kernel,
        grid_spec=pltpu.PrefetchScalarGridSpec(
            num_scalar_prefetch=1,
            grid=(K,),
            in_specs=[
                pl.BlockSpec((B, 1, 16, 128), lambda i, idx_ref: (0, idx_ref[i], 0, 0)),
            ],
            out_specs=pl.BlockSpec((1, 16, 128), lambda i, idx_ref: (i, 0, 0)),
        ),
        out_shape=jax.ShapeDtypeStruct((K, 16, 128), jnp.float32),
    )(topk_indices, h4)
    return out.reshape(K, D)


def kernel(hidden_states, memory):
    importance = _importance(hidden_states)
    _, topk_indices = jax.lax.top_k(importance, K)
    return _gather_mean(hidden_states, topk_indices)


# speedup vs baseline: 2.7178x; 2.7178x over previous
import functools

import jax
import jax.numpy as jnp
from jax import lax
from jax.experimental import pallas as pl
from jax.experimental.pallas import tpu as pltpu
from jax.experimental.pallas import tpu_sc as plsc

B = 4
S = 4096
D = 2048
K = 512
SBLK = 256


def _norm_kernel(x_ref, imp_ref, hm_ref):
    x = x_ref[...]
    xc = x.reshape(B, SBLK, 16, 128)
    acc = xc[:, :, 0, :] * xc[:, :, 0, :]
    for c in range(1, 16):
        acc = acc + xc[:, :, c, :] * xc[:, :, c, :]
    r = acc
    s2 = acc
    for j in range(1, 16):
        r = pltpu.roll(r, 120, axis=2)
        s2 = s2 + r
    t1 = s2 + pltpu.roll(s2, 124, axis=2)
    t2 = t1 + pltpu.roll(t1, 126, axis=2)
    t3 = t2 + pltpu.roll(t2, 127, axis=2)
    ss = t3[:, :, 0]
    n = jnp.sqrt(ss)
    imp_ref[...] = jnp.mean(n, axis=0)
    hm_ref[...] = jnp.mean(x, axis=0)


def _norm_pass(hidden_states):
    return pl.pallas_call(
        _norm_kernel,
        grid=(S // SBLK,),
        in_specs=[pl.BlockSpec((B, SBLK, D), lambda i: (0, i, 0))],
        out_specs=[pl.BlockSpec((SBLK,), lambda i: (i,)),
                   pl.BlockSpec((SBLK, D), lambda i: (i, 0))],
        out_shape=[jax.ShapeDtypeStruct((S,), jnp.float32),
                   jax.ShapeDtypeStruct((S, D), jnp.float32)],
    )(hidden_states)


def _make_sc_gather():
    info = plsc.get_sparse_core_info()
    nc, ns = info.num_cores, info.num_subcores
    nw = nc * ns
    b_per_w = K // nw
    mesh = plsc.VectorSubcoreMesh(core_axis_name="c", subcore_axis_name="s")

    @functools.partial(
        pl.kernel, mesh=mesh,
        out_type=jax.ShapeDtypeStruct((K, D), jnp.float32),
        scratch_types=[
            pltpu.VMEM((b_per_w,), jnp.int32),
            pltpu.VMEM((b_per_w, D), jnp.float32),
            pltpu.SemaphoreType.DMA,
        ],
    )
    def sc_gather(hmean_hbm, idx_hbm, out_hbm, idx_v, rows_v, sem):
        wid = lax.axis_index("s") * nc + lax.axis_index("c")
        base = wid * b_per_w
        pltpu.sync_copy(idx_hbm.at[pl.ds(base, b_per_w)], idx_v)
        pltpu.async_copy(hmean_hbm.at[idx_v], rows_v, sem).wait()
        pltpu.sync_copy(rows_v, out_hbm.at[pl.ds(base, b_per_w)])

    return sc_gather


def kernel(hidden_states, memory):
    importance, hmean = _norm_pass(hidden_states)
    _, topk_indices = jax.lax.top_k(importance, K)
    gather = _make_sc_gather()
    return gather(hmean, topk_indices)


# speedup vs baseline: 7.4935x; 2.7572x over previous
import functools

import jax
import jax.numpy as jnp
from jax import lax
from jax.experimental import pallas as pl
from jax.experimental.pallas import tpu as pltpu
from jax.experimental.pallas import tpu_sc as plsc

B = 4
S = 4096
D = 2048
K = 512
SBLK = 256


def _norm_kernel(x_ref, imp_ref, hm_ref):
    x = x_ref[...]
    c0 = x[:, :, 0:128]
    acc = c0 * c0
    for c in range(1, 16):
        xc = x[:, :, c * 128:(c + 1) * 128]
        acc = acc + xc * xc
    s2 = acc
    for j in range(1, 16):
        s2 = s2 + pltpu.roll(acc, 128 - 8 * j, axis=2)
    t1 = s2 + pltpu.roll(s2, 124, axis=2)
    t2 = t1 + pltpu.roll(t1, 126, axis=2)
    t3 = t2 + pltpu.roll(t2, 127, axis=2)
    ss = t3[:, :, 0]
    n = jnp.sqrt(ss)
    imp_ref[...] = jnp.mean(n, axis=0)
    hm_ref[...] = jnp.mean(x, axis=0)


def _norm_pass(hidden_states):
    return pl.pallas_call(
        _norm_kernel,
        grid=(S // SBLK,),
        in_specs=[pl.BlockSpec((B, SBLK, D), lambda i: (0, i, 0))],
        out_specs=[pl.BlockSpec((SBLK,), lambda i: (i,)),
                   pl.BlockSpec((SBLK, D), lambda i: (i, 0))],
        out_shape=[jax.ShapeDtypeStruct((S,), jnp.float32),
                   jax.ShapeDtypeStruct((S, D), jnp.float32)],
    )(hidden_states)


def _make_sc_gather():
    info = plsc.get_sparse_core_info()
    nc, ns = info.num_cores, info.num_subcores
    nw = nc * ns
    b_per_w = K // nw
    mesh = plsc.VectorSubcoreMesh(core_axis_name="c", subcore_axis_name="s")

    @functools.partial(
        pl.kernel, mesh=mesh,
        out_type=jax.ShapeDtypeStruct((K, D), jnp.float32),
        scratch_types=[
            pltpu.VMEM((b_per_w,), jnp.int32),
            pltpu.VMEM((b_per_w, D), jnp.float32),
            pltpu.SemaphoreType.DMA,
        ],
    )
    def sc_gather(hmean_hbm, idx_hbm, out_hbm, idx_v, rows_v, sem):
        wid = lax.axis_index("s") * nc + lax.axis_index("c")
        base = wid * b_per_w
        pltpu.sync_copy(idx_hbm.at[pl.ds(base, b_per_w)], idx_v)
        pltpu.async_copy(hmean_hbm.at[idx_v], rows_v, sem).wait()
        pltpu.sync_copy(rows_v, out_hbm.at[pl.ds(base, b_per_w)])

    return sc_gather


def kernel(hidden_states, memory):
    importance, hmean = _norm_pass(hidden_states)
    _, topk_indices = jax.lax.top_k(importance, K)
    gather = _make_sc_gather()
    return gather(hmean, topk_indices)
